# baseline (speedup 1.0000x reference)
import functools

import jax
import jax.numpy as jnp
from jax import lax
from jax.experimental import pallas as pl
from jax.experimental.pallas import tpu as pltpu
from jax.experimental.pallas import tpu_sc as plsc

MBITS = 8
SHIFT = 23 - MBITS
EXPLO = 107
NEXP = 36
NB = NEXP << MBITS
KEY_BIAS = EXPLO << MBITS
HIST = 2 * NB + 64
HBASE = 64
GROUP = 4
QTR = NB // GROUP
L = 16


def _build(n_per_sample, chunk, unroll):
    vpc = chunk // L
    nchunk = n_per_sample // (GROUP * chunk)
    mesh = plsc.VectorSubcoreMesh(core_axis_name="c", subcore_axis_name="s",
                                  num_cores=2, num_subcores=16)

    @functools.partial(
        pl.kernel,
        out_type=jax.ShapeDtypeStruct((32, L), jnp.float32),
        mesh=mesh,
        compiler_params=pltpu.CompilerParams(needs_layout_passes=False,
                                             use_tc_tiling_on_sc=True),
        scratch_types=[
            pltpu.VMEM((chunk // 512, 512), jnp.float32),
            pltpu.VMEM((chunk // 512, 512), jnp.float32),
            pltpu.VMEM((chunk // 512, 512), jnp.int32),
            pltpu.VMEM((chunk // 512, 512), jnp.int32),
            pltpu.SemaphoreType.DMA,
            pltpu.SemaphoreType.DMA,
            pltpu.VMEM((HIST,), jnp.int32),
            pltpu.VMEM((QTR,), jnp.int32),
            pltpu.VMEM((QTR,), jnp.int32),
            pltpu.VMEM((QTR,), jnp.int32),
            pltpu.VMEM((QTR,), jnp.int32),
            pltpu.VMEM((QTR,), jnp.int32),
            pltpu.VMEM((QTR,), jnp.int32),
            pltpu.VMEM((L,), jnp.float32),
            pltpu.VMEM((L,), jnp.int32),
            pltpu.VMEM((GROUP * L,), jnp.int32),
            pltpu.VMEM_SHARED((16 * HIST,), jnp.int32),
            pltpu.VMEM_SHARED((16 * L,), jnp.int32),
            pltpu.VMEM_SHARED((16 * L,), jnp.int32),
            pltpu.VMEM_SHARED((16 * L,), jnp.int32),
        ],
    )
    def sc_kernel(logits_hbm, targets_hbm, out_hbm, lbuf0, lbuf1, tbuf0,
                  tbuf1, sem0, sem1, hist, accq, accp, tmp0, tmp1, tmp2,
                  tmp3, iobuf, sbuf, sbuf4, sh_hist, sh_p, sh_q, sh_pp):
        lbufs, tbufs, sems = (lbuf0, lbuf1), (tbuf0, tbuf1), (sem0, sem1)
        tmps = (tmp0, tmp1, tmp2, tmp3)
        c = lax.axis_index("c")
        s = lax.axis_index("s")
        g = s // GROUP
        q = s % GROUP
        sample = c * GROUP + g
        ebase = sample * n_per_sample + q * (n_per_sample // GROUP)
        iota = lax.iota(jnp.int32, L)
        zero16 = jnp.zeros((L,), jnp.int32)
        ones = jnp.full((L,), 1, jnp.int32)

        def zbody(i):
            hist[pl.ds(i * L, L)] = zero16
        plsc.parallel_loop(0, HIST // L, 1, unroll=8)(zbody)

        rows = chunk // 512
        rbase0 = q * (n_per_sample // GROUP // 512)

        def issue(ck, b):
            r0 = rbase0 + ck * rows
            pltpu.async_copy(logits_hbm.at[sample, 0, pl.ds(r0, rows), :],
                             lbufs[b], sems[b])
            pltpu.async_copy(targets_hbm.at[sample, pl.ds(r0, rows), :],
                             tbufs[b], sems[b])

        def drain(ck, b):
            r0 = rbase0 + ck * rows
            pltpu.make_async_copy(logits_hbm.at[sample, 0, pl.ds(r0, rows), :],
                                  lbufs[b], sems[b]).wait()
            pltpu.make_async_copy(targets_hbm.at[sample, pl.ds(r0, rows), :],
                                  tbufs[b], sems[b]).wait()

        issue(0, 0)
        issue(1, 1)

        def pair_body(ci, _):
            for b in range(2):
                ck = ci * 2 + b
                drain(ck, b)
                lbuf, tbuf = lbufs[b], tbufs[b]

                def vec_body(vi):
                    base = vi * L
                    lv = lbuf[base // 512, pl.ds(base % 512, L)]
                    tv = tbuf[base // 512, pl.ds(base % 512, L)]
                    flipped = lax.bitcast_convert_type(
                        lax.bitcast_convert_type(lv, jnp.int32)
                        ^ (tv << 31), jnp.float32)
                    err = 1.0 + flipped
                    bits = lax.bitcast_convert_type(err, jnp.int32)
                    cls = tv * NB
                    raw = (bits >> SHIFT) - (KEY_BIAS - HBASE) + cls
                    k = jnp.minimum(jnp.maximum(raw, (tv << 4) + iota),
                                    cls + (HBASE + NB - 1))
                    plsc.addupdate_scatter(hist, [k], ones)
                plsc.parallel_loop(0, vpc, 1, unroll=unroll)(vec_body)

                @pl.when(ck + 2 < nchunk)
                def _():
                    issue(ck + 2, b)
            return 0

        lax.fori_loop(0, nchunk // 2, pair_body, 0)
        pacc = hist[pl.ds(L, L)]

        seg = HIST // 4
        for si in range(4):
            pltpu.async_copy(hist.at[pl.ds(si * seg, seg)],
                             sh_hist.at[pl.ds(s * HIST + si * seg, seg)],
                             sem1)
        sbuf[...] = pacc
        pltpu.async_copy(sbuf, sh_p.at[pl.ds(s * L, L)], sem1)
        for si in range(4):
            pltpu.make_async_copy(hist.at[pl.ds(si * seg, seg)],
                                  sh_hist.at[pl.ds(s * HIST + si * seg, seg)],
                                  sem1).wait()
        pltpu.make_async_copy(sbuf, sh_p.at[pl.ds(s * L, L)], sem1).wait()
        plsc.subcore_barrier()

        def combine(cls, dst):
            wbase = HBASE + cls * NB + q * QTR
            for j in range(GROUP):
                other = g * GROUP + j
                pltpu.async_copy(
                    sh_hist.at[pl.ds(other * HIST + wbase, QTR)],
                    tmps[j], sem0)
            for j in range(GROUP):
                other = g * GROUP + j
                pltpu.make_async_copy(
                    sh_hist.at[pl.ds(other * HIST + wbase, QTR)],
                    tmps[j], sem0).wait()

            def body(i, acc):
                d = pl.ds(i * L, L)
                v = ((tmps[0][d] + tmps[1][d])
                     + (tmps[2][d] + tmps[3][d]))
                dst[d] = v
                return acc + v
            return plsc.parallel_loop(
                0, QTR // L, 1, unroll=4, carry=zero16)(body)

        qsv = combine(0, accq)
        psv = combine(1, accp)

        sbuf[...] = qsv
        pltpu.sync_copy(sbuf, sh_q.at[pl.ds(s * L, L)])
        sbuf[...] = psv
        pltpu.sync_copy(sbuf, sh_pp.at[pl.ds(s * L, L)])
        plsc.subcore_barrier()

        gb = g * GROUP * L
        pltpu.sync_copy(sh_p.at[pl.ds(gb, GROUP * L)], sbuf4)
        p_vec = (sbuf4[pl.ds(0, L)] + sbuf4[pl.ds(L, L)]
                 + sbuf4[pl.ds(2 * L, L)] + sbuf4[pl.ds(3 * L, L)])
        p_i0 = jnp.sum(p_vec)
        offq = jnp.int32(0)
        offp = jnp.int32(0)
        qtot = jnp.int32(0)
        ptot = jnp.int32(0)
        pltpu.sync_copy(sh_q.at[pl.ds(gb, GROUP * L)], sbuf4)
        qjs = [jnp.sum(sbuf4[pl.ds(j * L, L)]) for j in range(GROUP)]
        pltpu.sync_copy(sh_pp.at[pl.ds(gb, GROUP * L)], sbuf4)
        pjs = [jnp.sum(sbuf4[pl.ds(j * L, L)]) for j in range(GROUP)]
        for j in range(GROUP):
            sel = jnp.where(jnp.int32(j) < q, jnp.int32(1), jnp.int32(0))
            offq = offq + sel * qjs[j]
            offp = offp + sel * pjs[j]
            qtot = qtot + qjs[j]
            ptot = ptot + pjs[j]
        p_i = p_i0 + ptot
        p_f = p_i.astype(jnp.float32)
        qtot_f = qtot.astype(jnp.float32)
        ptot_f = ptot.astype(jnp.float32)
        one = jnp.float32(1.0)

        kbase = q * QTR + KEY_BIAS
        lane15 = jnp.full((L,), L - 1, jnp.int32)

        def term_body(i, carry):
            cq_c, cp_c, acc = carry
            for u in range(2):
                idx = i * 2 + u
                qv_i = accq[pl.ds(idx * L, L)]
                pv_i = accp[pl.ds(idx * L, L)]
                cq_i = plsc.cumsum(qv_i) + cq_c
                cp_i = plsc.cumsum(pv_i) + cp_c
                qvf = qv_i.astype(jnp.float32)
                pvf = pv_i.astype(jnp.float32)
                cqf = cq_i.astype(jnp.float32)
                cpf = cp_i.astype(jnp.float32)
                n0 = qtot_f - cqf
                d0 = jnp.maximum(p_f + n0, one)
                d1 = jnp.maximum(p_f + n0 + qvf, one)
                pm = p_f - ptot_f + cpf - pvf
                vbits = ((kbase + idx * L + iota) << SHIFT) | (1 << (SHIFT - 1))
                vhat = lax.bitcast_convert_type(vbits, jnp.float32)
                term = vhat * (pvf / d0 + pm * qvf / (d0 * d1))
                is_top = (p_i == 0) & (n0 == jnp.float32(0.0)) & (qv_i > 0)
                acc = acc + term + jnp.where(is_top, vhat, jnp.float32(0.0))
                cq_c = cq_i.at[lane15].get(mode="promise_in_bounds")
                cp_c = cp_i.at[lane15].get(mode="promise_in_bounds")
            return (cq_c, cp_c, acc)

        zf16 = jnp.zeros((L,), jnp.float32)
        _, _, acc = lax.fori_loop(
            0, QTR // (2 * L), term_body,
            (zero16 + offq, zero16 + offp, zf16))
        qloss = jnp.sum(acc)
        iobuf[...] = jnp.where(iota == 0, qloss, jnp.float32(0.0))
        wid = c * 16 + s
        pltpu.sync_copy(iobuf, out_hbm.at[wid])

    return sc_kernel


_sc_cache = {}


def _get_sc_kernel():
    if "k" not in _sc_cache:
        _sc_cache["k"] = _build(n_per_sample=512 * 512, chunk=4096, unroll=6)
    return _sc_cache["k"]


def kernel(logits, targets):
    out = _get_sc_kernel()(logits, targets)
    return out.sum() / jnp.float32(logits.shape[0])


# speedup vs baseline: 1.0351x; 1.0351x over previous
import functools

import jax
import jax.numpy as jnp
from jax import lax
from jax.experimental import pallas as pl
from jax.experimental.pallas import tpu as pltpu
from jax.experimental.pallas import tpu_sc as plsc

MBITS = 8
SHIFT = 23 - MBITS
EXPLO = 107
NEXP = 36
NB = NEXP << MBITS
KEY_BIAS = EXPLO << MBITS
HIST = 2 * NB + 64
HBASE = 64
GROUP = 4
QTR = NB // GROUP
L = 16


def _build(n_per_sample, chunk, unroll):
    vpc = chunk // L
    nchunk = n_per_sample // (GROUP * chunk)
    mesh = plsc.VectorSubcoreMesh(core_axis_name="c", subcore_axis_name="s",
                                  num_cores=2, num_subcores=16)

    @functools.partial(
        pl.kernel,
        out_type=jax.ShapeDtypeStruct((32, L), jnp.float32),
        mesh=mesh,
        compiler_params=pltpu.CompilerParams(needs_layout_passes=False,
                                             use_tc_tiling_on_sc=True),
        scratch_types=[
            pltpu.VMEM((chunk // 512, 512), jnp.float32),
            pltpu.VMEM((chunk // 512, 512), jnp.float32),
            pltpu.VMEM((chunk // 512, 512), jnp.int32),
            pltpu.VMEM((chunk // 512, 512), jnp.int32),
            pltpu.SemaphoreType.DMA,
            pltpu.SemaphoreType.DMA,
            pltpu.VMEM((HIST,), jnp.int32),
            pltpu.VMEM((QTR,), jnp.int32),
            pltpu.VMEM((QTR,), jnp.int32),
            pltpu.VMEM((QTR,), jnp.int32),
            pltpu.VMEM((QTR,), jnp.int32),
            pltpu.VMEM((QTR,), jnp.int32),
            pltpu.VMEM((QTR,), jnp.int32),
            pltpu.VMEM((L,), jnp.float32),
            pltpu.VMEM((L,), jnp.int32),
            pltpu.VMEM((GROUP * L,), jnp.int32),
            pltpu.VMEM_SHARED((16 * HIST,), jnp.int32),
            pltpu.VMEM_SHARED((16 * L,), jnp.int32),
            pltpu.VMEM_SHARED((16 * L,), jnp.int32),
            pltpu.VMEM_SHARED((16 * L,), jnp.int32),
        ],
    )
    def sc_kernel(logits_hbm, targets_hbm, out_hbm, lbuf0, lbuf1, tbuf0,
                  tbuf1, sem0, sem1, hist, accq, accp, tmp0, tmp1, tmp2,
                  tmp3, iobuf, sbuf, sbuf4, sh_hist, sh_p, sh_q, sh_pp):
        lbufs, tbufs, sems = (lbuf0, lbuf1), (tbuf0, tbuf1), (sem0, sem1)
        tmps = (tmp0, tmp1, tmp2, tmp3)
        c = lax.axis_index("c")
        s = lax.axis_index("s")
        g = s // GROUP
        q = s % GROUP
        sample = c * GROUP + g
        ebase = sample * n_per_sample + q * (n_per_sample // GROUP)
        iota = lax.iota(jnp.int32, L)
        zero16 = jnp.zeros((L,), jnp.int32)
        ones = jnp.full((L,), 1, jnp.int32)

        def zbody(i):
            hist[pl.ds(i * L, L)] = zero16
        plsc.parallel_loop(0, HIST // L, 1, unroll=8)(zbody)

        rows = chunk // 512
        rbase0 = q * (n_per_sample // GROUP // 512)

        def issue(ck, b):
            r0 = rbase0 + ck * rows
            pltpu.async_copy(logits_hbm.at[sample, 0, pl.ds(r0, rows), :],
                             lbufs[b], sems[b])
            pltpu.async_copy(targets_hbm.at[sample, pl.ds(r0, rows), :],
                             tbufs[b], sems[b])

        def drain(ck, b):
            r0 = rbase0 + ck * rows
            pltpu.make_async_copy(logits_hbm.at[sample, 0, pl.ds(r0, rows), :],
                                  lbufs[b], sems[b]).wait()
            pltpu.make_async_copy(targets_hbm.at[sample, pl.ds(r0, rows), :],
                                  tbufs[b], sems[b]).wait()

        issue(0, 0)
        issue(1, 1)

        def pair_body(ci, _):
            for b in range(2):
                ck = ci * 2 + b
                drain(ck, b)
                lbuf, tbuf = lbufs[b], tbufs[b]

                def vec_body(vi):
                    base = vi * L
                    lv = lbuf[base // 512, pl.ds(base % 512, L)]
                    tv = tbuf[base // 512, pl.ds(base % 512, L)]
                    flipped = lax.bitcast_convert_type(
                        lax.bitcast_convert_type(lv, jnp.int32)
                        ^ (tv << 31), jnp.float32)
                    err = 1.0 + flipped
                    bits = lax.bitcast_convert_type(err, jnp.int32)
                    cls = tv * NB
                    raw = (bits >> SHIFT) - (KEY_BIAS - HBASE) + cls
                    k = jnp.minimum(jnp.maximum(raw, (tv << 4) + iota),
                                    cls + (HBASE + NB - 1))
                    plsc.addupdate_scatter(hist, [k], ones)
                plsc.parallel_loop(0, vpc, 1, unroll=unroll)(vec_body)

                @pl.when(ck + 2 < nchunk)
                def _():
                    issue(ck + 2, b)
            return 0

        lax.fori_loop(0, nchunk // 2, pair_body, 0)
        pacc = hist[pl.ds(L, L)]

        seg = HIST // 4
        for si in range(4):
            pltpu.async_copy(hist.at[pl.ds(si * seg, seg)],
                             sh_hist.at[pl.ds(s * HIST + si * seg, seg)],
                             sem1)
        sbuf[...] = pacc
        pltpu.async_copy(sbuf, sh_p.at[pl.ds(s * L, L)], sem1)
        for si in range(4):
            pltpu.make_async_copy(hist.at[pl.ds(si * seg, seg)],
                                  sh_hist.at[pl.ds(s * HIST + si * seg, seg)],
                                  sem1).wait()
        pltpu.make_async_copy(sbuf, sh_p.at[pl.ds(s * L, L)], sem1).wait()
        plsc.subcore_barrier()

        def combine(cls, dst):
            wbase = HBASE + cls * NB + q * QTR
            for j in range(GROUP):
                other = g * GROUP + j
                pltpu.async_copy(
                    sh_hist.at[pl.ds(other * HIST + wbase, QTR)],
                    tmps[j], sem0)
            for j in range(GROUP):
                other = g * GROUP + j
                pltpu.make_async_copy(
                    sh_hist.at[pl.ds(other * HIST + wbase, QTR)],
                    tmps[j], sem0).wait()

            def body(i, acc):
                d = pl.ds(i * L, L)
                v = ((tmps[0][d] + tmps[1][d])
                     + (tmps[2][d] + tmps[3][d]))
                dst[d] = v
                return acc + v
            return plsc.parallel_loop(
                0, QTR // L, 1, unroll=4, carry=zero16)(body)

        qsv = combine(0, accq)
        psv = combine(1, accp)

        sbuf[...] = qsv
        pltpu.sync_copy(sbuf, sh_q.at[pl.ds(s * L, L)])
        sbuf[...] = psv
        pltpu.sync_copy(sbuf, sh_pp.at[pl.ds(s * L, L)])
        plsc.subcore_barrier()

        gb = g * GROUP * L
        pltpu.sync_copy(sh_p.at[pl.ds(gb, GROUP * L)], sbuf4)
        p_vec = (sbuf4[pl.ds(0, L)] + sbuf4[pl.ds(L, L)]
                 + sbuf4[pl.ds(2 * L, L)] + sbuf4[pl.ds(3 * L, L)])
        p_i0 = jnp.sum(p_vec)
        offq = jnp.int32(0)
        offp = jnp.int32(0)
        qtot = jnp.int32(0)
        ptot = jnp.int32(0)
        pltpu.sync_copy(sh_q.at[pl.ds(gb, GROUP * L)], sbuf4)
        qjs = [jnp.sum(sbuf4[pl.ds(j * L, L)]) for j in range(GROUP)]
        pltpu.sync_copy(sh_pp.at[pl.ds(gb, GROUP * L)], sbuf4)
        pjs = [jnp.sum(sbuf4[pl.ds(j * L, L)]) for j in range(GROUP)]
        for j in range(GROUP):
            sel = jnp.where(jnp.int32(j) < q, jnp.int32(1), jnp.int32(0))
            offq = offq + sel * qjs[j]
            offp = offp + sel * pjs[j]
            qtot = qtot + qjs[j]
            ptot = ptot + pjs[j]
        p_i = p_i0 + ptot
        p_f = p_i.astype(jnp.float32)
        qtot_f = qtot.astype(jnp.float32)
        ptot_f = ptot.astype(jnp.float32)
        one = jnp.float32(1.0)

        kbase = q * QTR + KEY_BIAS
        lane15 = jnp.full((L,), L - 1, jnp.int32)

        def term_body(i, carry):
            cq_c, cp_c, acc = carry
            for u in range(2):
                idx = i * 2 + u
                qv_i = accq[pl.ds(idx * L, L)]
                pv_i = accp[pl.ds(idx * L, L)]
                cq_i = plsc.cumsum(qv_i) + cq_c
                cp_i = plsc.cumsum(pv_i) + cp_c
                qvf = qv_i.astype(jnp.float32)
                pvf = pv_i.astype(jnp.float32)
                cqf = cq_i.astype(jnp.float32)
                cpf = cp_i.astype(jnp.float32)
                n0 = qtot_f - cqf
                d0 = jnp.maximum(p_f + n0, one)
                d1 = jnp.maximum(p_f + n0 + qvf, one)
                pm = p_f - ptot_f + cpf - pvf
                vbits = ((kbase + idx * L + iota) << SHIFT) | (1 << (SHIFT - 1))
                vhat = lax.bitcast_convert_type(vbits, jnp.float32)
                term = vhat * (pvf / d0 + pm * qvf / (d0 * d1))
                is_top = (p_i == 0) & (n0 == jnp.float32(0.0)) & (qv_i > 0)
                acc = acc + term + jnp.where(is_top, vhat, jnp.float32(0.0))
                cq_c = cq_i.at[lane15].get(mode="promise_in_bounds")
                cp_c = cp_i.at[lane15].get(mode="promise_in_bounds")
            return (cq_c, cp_c, acc)

        zf16 = jnp.zeros((L,), jnp.float32)
        _, _, acc = lax.fori_loop(
            0, QTR // (2 * L), term_body,
            (zero16 + offq, zero16 + offp, zf16))
        qloss = jnp.sum(acc)
        iobuf[...] = jnp.where(iota == 0, qloss, jnp.float32(0.0))
        wid = c * 16 + s
        pltpu.sync_copy(iobuf, out_hbm.at[wid])

    return sc_kernel


_sc_cache = {}


def _get_sc_kernel():
    if "k" not in _sc_cache:
        _sc_cache["k"] = _build(n_per_sample=512 * 512, chunk=8192, unroll=4)
    return _sc_cache["k"]


def kernel(logits, targets):
    out = _get_sc_kernel()(logits, targets)
    return out.sum() / jnp.float32(logits.shape[0])
